# baseline (device time: 54251 ns/iter reference)
import jax
import jax.numpy as jnp
from jax import lax
from jax.experimental import pallas as pl
from jax.experimental.pallas import tpu as pltpu

N_DEV = 32
B, SQ, D_MODEL, HQ, DH = 2, 512, 768, 8, 64
DQK = HQ * DH
COLS = DQK + HQ
PH = 4
PROWS = B * SQ // PH
CHUNK = PROWS // N_DEV

_MESH = pl.DeviceIdType.MESH


def _regroup(a):
    return jnp.swapaxes(a.reshape(2, 4, 64, DH), 0, 1).reshape(4, 128, DH)


def kernel(x, Wq, K_ext, V_ext, Wo):
    def body(x_ref, wq_ref, k_ref, v_ref, wo_ref, out_ref,
             acc_ref, sendbuf_ref, stage_ref, ostage_ref,
             send1, recv1, send2, recv2):
        d = lax.axis_index("i")

        barrier_sem = pltpu.get_barrier_semaphore()
        for o in range(1, N_DEV):
            t = lax.rem(d + o, N_DEV)
            pl.semaphore_signal(barrier_sem, inc=1, device_id=(t,),
                                device_id_type=_MESH)
        pl.semaphore_wait(barrier_sem, N_DEV - 1)

        r1 = [[] for _ in range(PH)]
        r2 = [[] for _ in range(PH)]
        wob = wo_ref[...].astype(jnp.bfloat16)

        def compute_phase(p):
            b, hf = p // 2, p % 2
            base = p * PROWS
            qm = jnp.dot(x_ref[b, hf * PROWS:(hf + 1) * PROWS, :],
                         wq_ref[...], preferred_element_type=jnp.float32)
            l_cols = []
            for h in range(HQ):
                qg = qm[:, h * DH:(h + 1) * DH].reshape(4, 64, DH)
                kg = _regroup(k_ref[b, :, h, :])
                vg = _regroup(v_ref[b, :, h, :])
                sc = lax.dot_general(
                    qg, kg, (((2,), (2,)), ((0,), (0,))),
                    preferred_element_type=jnp.float32) * 0.125
                w = jnp.exp(sc)
                l_cols.append(
                    jnp.sum(w, axis=2, keepdims=True).reshape(PROWS, 1))
                og = lax.dot_general(
                    w, vg, (((2,), (1,)), ((0,), (0,))),
                    preferred_element_type=jnp.float32)
                acc_ref[base:base + PROWS,
                        h * DH:(h + 1) * DH] = og.reshape(PROWS, DH)
            acc_ref[base:base + PROWS, DQK:] = jnp.concatenate(
                l_cols, axis=1)
            sendbuf_ref[base:base + PROWS, :] = acc_ref[
                base:base + PROWS, :].astype(jnp.bfloat16)
            for o in range(1, N_DEV):
                t = lax.rem(d + o, N_DEV)
                r = pltpu.make_async_remote_copy(
                    src_ref=sendbuf_ref.at[
                        pl.ds(base + t * CHUNK, CHUNK), :],
                    dst_ref=stage_ref.at[p, o],
                    send_sem=send1.at[p, o],
                    recv_sem=recv1.at[p, o],
                    device_id=(t,),
                    device_id_type=_MESH,
                )
                r.start()
                r1[p].append(r)

        def finish_phase(p):
            b, hf = p // 2, p % 2
            for r in r1[p]:
                r.wait()
            rows = pl.ds(p * PROWS + d * CHUNK, CHUNK)
            red = acc_ref[rows, :] + jnp.sum(
                stage_ref[p, 1:, :, :].astype(jnp.float32), axis=0)
            ctx = jnp.concatenate(
                [red[:, h * DH:(h + 1) * DH] / red[:, DQK + h:DQK + h + 1]
                 for h in range(HQ)], axis=1)
            orows = pl.ds(hf * PROWS + d * CHUNK, CHUNK)
            ostage_ref[b, orows, :] = ctx.astype(jnp.bfloat16)
            for o in range(1, N_DEV):
                t = lax.rem(d + o, N_DEV)
                r = pltpu.make_async_remote_copy(
                    src_ref=ostage_ref.at[b, orows, :],
                    dst_ref=ostage_ref.at[b, orows, :],
                    send_sem=send2.at[p, o],
                    recv_sem=recv2.at[p, o],
                    device_id=(t,),
                    device_id_type=_MESH,
                )
                r.start()
                r2[p].append(r)

        def deliver_phase(p):
            b, hf = p // 2, p % 2
            for r in r2[p]:
                r.wait()
            out_ref[b, hf * PROWS:(hf + 1) * PROWS, :] = jnp.dot(
                ostage_ref[b, hf * PROWS:(hf + 1) * PROWS, :], wob,
                preferred_element_type=jnp.float32)

        for p in range(PH):
            compute_phase(p)
            if p >= 1:
                finish_phase(p - 1)
            if p >= 2:
                deliver_phase(p - 2)
        finish_phase(PH - 1)
        deliver_phase(PH - 2)
        deliver_phase(PH - 1)

    return pl.pallas_call(
        body,
        out_shape=jax.ShapeDtypeStruct((B, SQ, D_MODEL), jnp.float32),
        in_specs=[pl.BlockSpec(memory_space=pltpu.VMEM)] * 5,
        out_specs=pl.BlockSpec(memory_space=pltpu.VMEM),
        scratch_shapes=[
            pltpu.VMEM((B * SQ, COLS), jnp.float32),
            pltpu.VMEM((B * SQ, COLS), jnp.bfloat16),
            pltpu.VMEM((PH, N_DEV, CHUNK, COLS), jnp.bfloat16),
            pltpu.VMEM((B, SQ, DQK), jnp.bfloat16),
            pltpu.SemaphoreType.DMA((PH, N_DEV)),
            pltpu.SemaphoreType.DMA((PH, N_DEV)),
            pltpu.SemaphoreType.DMA((PH, N_DEV)),
            pltpu.SemaphoreType.DMA((PH, N_DEV)),
        ],
        compiler_params=pltpu.CompilerParams(collective_id=0),
    )(x, Wq, K_ext, V_ext, Wo)


# device time: 50144 ns/iter; 1.0819x vs baseline; 1.0819x over previous
import jax
import jax.numpy as jnp
from jax import lax
from jax.experimental import pallas as pl
from jax.experimental.pallas import tpu as pltpu

N_DEV = 32
B, SQ, D_MODEL, HQ, DH = 2, 512, 768, 8, 64
DQK = HQ * DH
COLS = DQK + HQ
PH = 4
PROWS = B * SQ // PH
CHUNK = PROWS // N_DEV

_MESH = pl.DeviceIdType.MESH


def _regroup(a):
    return jnp.swapaxes(a.reshape(2, 4, 64, DH), 0, 1).reshape(4, 128, DH)


def kernel(x, Wq, K_ext, V_ext, Wo):
    def body(x_ref, wq_ref, k_ref, v_ref, wo_ref, out_ref,
             acc_ref, sendbuf_ref, stage_ref, ostage_ref,
             send1, recv1, send2, recv2):
        d = lax.axis_index("i")

        barrier_sem = pltpu.get_barrier_semaphore()
        for o in range(1, N_DEV):
            t = lax.rem(d + o, N_DEV)
            pl.semaphore_signal(barrier_sem, inc=1, device_id=(t,),
                                device_id_type=_MESH)
        pl.semaphore_wait(barrier_sem, N_DEV - 1)

        r1 = [[] for _ in range(PH)]
        r2 = [[] for _ in range(PH)]
        wob = wo_ref[...].astype(jnp.bfloat16)
        kv_cache = {}

        def kv_grouped(b, h):
            if (b, h) not in kv_cache:
                kv_cache[(b, h)] = (_regroup(k_ref[b, :, h, :]),
                                    _regroup(v_ref[b, :, h, :]))
            return kv_cache[(b, h)]

        def compute_phase(p):
            b, hf = p // 2, p % 2
            base = p * PROWS
            qm = jnp.dot(x_ref[b, hf * PROWS:(hf + 1) * PROWS, :],
                         wq_ref[...], preferred_element_type=jnp.float32)
            l_cols = []
            for h in range(HQ):
                qg = qm[:, h * DH:(h + 1) * DH].reshape(4, 64, DH)
                kg, vg = kv_grouped(b, h)
                sc = lax.dot_general(
                    qg, kg, (((2,), (2,)), ((0,), (0,))),
                    preferred_element_type=jnp.float32) * 0.125
                w = jnp.exp(sc)
                l_cols.append(
                    jnp.sum(w, axis=2, keepdims=True).reshape(PROWS, 1))
                og = lax.dot_general(
                    w, vg, (((2,), (1,)), ((0,), (0,))),
                    preferred_element_type=jnp.float32)
                acc_ref[base:base + PROWS,
                        h * DH:(h + 1) * DH] = og.reshape(PROWS, DH)
            acc_ref[base:base + PROWS, DQK:] = jnp.concatenate(
                l_cols, axis=1)
            sendbuf_ref[base:base + PROWS, :] = acc_ref[
                base:base + PROWS, :].astype(jnp.bfloat16)
            for o in range(1, N_DEV):
                t = lax.rem(d + o, N_DEV)
                r = pltpu.make_async_remote_copy(
                    src_ref=sendbuf_ref.at[
                        pl.ds(base + t * CHUNK, CHUNK), :],
                    dst_ref=stage_ref.at[p, o],
                    send_sem=send1.at[p, o],
                    recv_sem=recv1.at[p, o],
                    device_id=(t,),
                    device_id_type=_MESH,
                )
                r.start()
                r1[p].append(r)

        def finish_phase(p):
            b, hf = p // 2, p % 2
            for r in r1[p]:
                r.wait()
            rows = pl.ds(p * PROWS + d * CHUNK, CHUNK)
            red = acc_ref[rows, :] + jnp.sum(
                stage_ref[p, 1:, :, :].astype(jnp.float32), axis=0)
            ctx = jnp.concatenate(
                [red[:, h * DH:(h + 1) * DH] / red[:, DQK + h:DQK + h + 1]
                 for h in range(HQ)], axis=1)
            orows = pl.ds(hf * PROWS + d * CHUNK, CHUNK)
            ostage_ref[b, orows, :] = ctx.astype(jnp.bfloat16)
            for o in range(1, N_DEV):
                t = lax.rem(d + o, N_DEV)
                r = pltpu.make_async_remote_copy(
                    src_ref=ostage_ref.at[b, orows, :],
                    dst_ref=ostage_ref.at[b, orows, :],
                    send_sem=send2.at[p, o],
                    recv_sem=recv2.at[p, o],
                    device_id=(t,),
                    device_id_type=_MESH,
                )
                r.start()
                r2[p].append(r)

        def deliver_phase(p):
            b, hf = p // 2, p % 2
            for r in r2[p]:
                r.wait()
            out_ref[b, hf * PROWS:(hf + 1) * PROWS, :] = jnp.dot(
                ostage_ref[b, hf * PROWS:(hf + 1) * PROWS, :], wob,
                preferred_element_type=jnp.float32)

        for p in range(PH):
            compute_phase(p)
            if p >= 1:
                finish_phase(p - 1)
            if p >= 2:
                deliver_phase(p - 2)
        finish_phase(PH - 1)
        deliver_phase(PH - 2)
        deliver_phase(PH - 1)

    return pl.pallas_call(
        body,
        out_shape=jax.ShapeDtypeStruct((B, SQ, D_MODEL), jnp.float32),
        in_specs=[pl.BlockSpec(memory_space=pltpu.VMEM)] * 5,
        out_specs=pl.BlockSpec(memory_space=pltpu.VMEM),
        scratch_shapes=[
            pltpu.VMEM((B * SQ, COLS), jnp.float32),
            pltpu.VMEM((B * SQ, COLS), jnp.bfloat16),
            pltpu.VMEM((PH, N_DEV, CHUNK, COLS), jnp.bfloat16),
            pltpu.VMEM((B, SQ, DQK), jnp.bfloat16),
            pltpu.SemaphoreType.DMA((PH, N_DEV)),
            pltpu.SemaphoreType.DMA((PH, N_DEV)),
            pltpu.SemaphoreType.DMA((PH, N_DEV)),
            pltpu.SemaphoreType.DMA((PH, N_DEV)),
        ],
        compiler_params=pltpu.CompilerParams(collective_id=0),
    )(x, Wq, K_ext, V_ext, Wo)
